# Initial kernel scaffold; baseline (speedup 1.0000x reference)
import jax
import jax.numpy as jnp
from jax.experimental import pallas as pl


def kernel(nfeats, efeats, edge_index, W1_0, b1_0, W2_0, b2_0, W1_1, b1_1, W2_1, b2_1):
    raise NotImplementedError("write your pallas kernel here")


# speedup vs baseline: 4.5295x; 4.5295x over previous
import functools

import jax
import jax.numpy as jnp
from jax import lax
from jax.experimental import pallas as pl
from jax.experimental.pallas import tpu as pltpu
from jax.experimental.pallas import tpu_sc as plsc

N = 10000
E = 320000
D_IN = 128
D_E = 16
D_OUT = 128

NC = 2
NS = 16
NW = NC * NS

GROUP = 125
BLK = 8
NBLK = 10
GROUPS = BLK * NBLK
EPW = GROUPS * GROUP
N_PAD = 10240
ROWS_PT = N_PAD // NS

_sc_mesh = plsc.VectorSubcoreMesh(core_axis_name="c", subcore_axis_name="s")


@functools.partial(
    pl.kernel,
    out_type=jax.ShapeDtypeStruct((NC, N_PAD, D_E), jnp.float32),
    mesh=_sc_mesh,
    scratch_types=[
        pltpu.VMEM((GROUPS, GROUP), jnp.int32),
        pltpu.VMEM((BLK, GROUP, D_E), jnp.float32),
        pltpu.VMEM_SHARED((N_PAD, D_E), jnp.float32),
    ],
    compiler_params=pltpu.CompilerParams(use_tc_tiling_on_sc=False),
)
def _sc_segment_sum(dst_hbm, ef_hbm, zeros_hbm, out_hbm, idx_v, buf_v, acc_sh):
    c = lax.axis_index("c")
    s = lax.axis_index("s")
    wid = c * NS + s

    r0 = s * ROWS_PT
    pltpu.sync_copy(zeros_hbm.at[pl.ds(r0, ROWS_PT)], acc_sh.at[pl.ds(r0, ROWS_PT)])
    plsc.subcore_barrier()

    pltpu.sync_copy(dst_hbm.at[wid], idx_v)

    def body(blk, carry):
        pltpu.sync_copy(ef_hbm.at[wid, blk], buf_v)
        for j in range(BLK):
            g = blk * BLK + j
            pltpu.sync_copy(buf_v.at[j], acc_sh.at[idx_v.at[g]], add=True)
        return carry

    lax.fori_loop(0, NBLK, body, 0)
    plsc.subcore_barrier()

    pltpu.sync_copy(acc_sh.at[pl.ds(r0, ROWS_PT)], out_hbm.at[c, pl.ds(r0, ROWS_PT)])


BN = 1000
NB = N // BN


def _mlp_body(nf_ref, part_ref,
              w1a0, w1b0, b10, w20, b20,
              w1a1, w1b1, b11, w21, b21,
              out_ref):
    hn = part_ref[0] + part_ref[1]
    nf = nf_ref[...]
    x = jnp.dot(nf, w1a0[...], preferred_element_type=jnp.float32)
    x = x + jnp.dot(hn, w1b0[...], preferred_element_type=jnp.float32)
    x = jnp.maximum(x + b10[...], 0.0)
    x = jnp.maximum(jnp.dot(x, w20[...], preferred_element_type=jnp.float32) + b20[...], 0.0)
    y = jnp.dot(x, w1a1[...], preferred_element_type=jnp.float32)
    y = y + jnp.dot(hn, w1b1[...], preferred_element_type=jnp.float32)
    y = jnp.maximum(y + b11[...], 0.0)
    y = jnp.maximum(jnp.dot(y, w21[...], preferred_element_type=jnp.float32) + b21[...], 0.0)
    out_ref[...] = y


def _row_spec(d):
    return pl.BlockSpec((BN, d), lambda i: (i, 0))


def _full_spec(*shape):
    return pl.BlockSpec(shape, lambda i: (0,) * len(shape))


_mlp_call = pl.pallas_call(
    _mlp_body,
    grid=(NB,),
    in_specs=[
        _row_spec(D_IN),
        pl.BlockSpec((NC, BN, D_E), lambda i: (0, i, 0)),
        _full_spec(D_IN, D_OUT), _full_spec(D_E, D_OUT), _full_spec(1, D_OUT),
        _full_spec(D_OUT, D_OUT), _full_spec(1, D_OUT),
        _full_spec(D_OUT, D_OUT), _full_spec(D_E, D_OUT), _full_spec(1, D_OUT),
        _full_spec(D_OUT, D_OUT), _full_spec(1, D_OUT),
    ],
    out_specs=_row_spec(D_OUT),
    out_shape=jax.ShapeDtypeStruct((N, D_OUT), jnp.float32),
)


@jax.jit
def kernel(nfeats, efeats, edge_index, W1_0, b1_0, W2_0, b2_0,
           W1_1, b1_1, W2_1, b2_1):
    dst = edge_index[1].reshape(NW, GROUPS, GROUP)
    ef = efeats.reshape(NW, NBLK, BLK, GROUP, D_E)
    zeros = jnp.zeros((N_PAD, D_E), jnp.float32)

    part = _sc_segment_sum(dst, ef, zeros)

    nf = nfeats.reshape(N, D_IN)
    out = _mlp_call(
        nf, part,
        W1_0[:D_IN], W1_0[D_IN:], b1_0.reshape(1, D_OUT),
        W2_0, b2_0.reshape(1, D_OUT),
        W1_1[:D_OUT], W1_1[D_OUT:], b1_1.reshape(1, D_OUT),
        W2_1, b2_1.reshape(1, D_OUT),
    )
    return out


# speedup vs baseline: 10.3913x; 2.2941x over previous
import functools

import jax
import jax.numpy as jnp
from jax import lax
from jax.experimental import pallas as pl
from jax.experimental.pallas import tpu as pltpu
from jax.experimental.pallas import tpu_sc as plsc

N = 10000
E = 320000
D_IN = 128
D_E = 16
D_OUT = 128

NC = 2
NS = 16
ETILES = E // 128
NQ = 4
FG = 8
ET_Q = ETILES // NQ
ET_CH = 125
NCH = ET_Q // ET_CH
N_PAD = 10240

_sc_mesh = plsc.VectorSubcoreMesh(core_axis_name="c", subcore_axis_name="s")


@functools.partial(
    pl.kernel,
    out_type=jax.ShapeDtypeStruct((NQ, D_E, N_PAD), jnp.float32),
    mesh=_sc_mesh,
    scratch_types=[
        pltpu.VMEM((2, N), jnp.float32),
        pltpu.VMEM((2, ET_CH, 2, 128), jnp.float32),
        pltpu.VMEM((2, ET_CH, 1, 128), jnp.int32),
        pltpu.SemaphoreType.DMA,
        pltpu.SemaphoreType.DMA,
        pltpu.SemaphoreType.DMA,
        pltpu.SemaphoreType.DMA,
    ],
    compiler_params=pltpu.CompilerParams(
        use_tc_tiling_on_sc=False, needs_layout_passes=False),
)
def _sc_segment_sum(ef_hbm, ei_hbm, out_hbm, acc, vals, idx,
                    sv0, sv1, si0, si1):
    c = lax.axis_index("c")
    s = lax.axis_index("s")
    wid = c * NS + s
    fg = wid % FG
    q = wid // FG
    half = fg // 4
    ds0 = (fg % 4) * 2

    zeros16 = jnp.zeros((16,), jnp.float32)

    def zero_body(i, carry):
        acc[0, pl.ds(i * 16, 16)] = zeros16
        acc[1, pl.ds(i * 16, 16)] = zeros16
        return carry

    lax.fori_loop(0, N // 16, zero_body, 0)

    sems_v = (sv0, sv1)
    sems_i = (si0, si1)

    def start(ch, b):
        et0 = q * ET_Q + ch * ET_CH
        cv = pltpu.async_copy(
            ef_hbm.at[half, pl.ds(et0, ET_CH), pl.ds(ds0, 2)],
            vals.at[b], sems_v[b])
        ci = pltpu.async_copy(
            ei_hbm.at[pl.ds(et0, ET_CH), pl.ds(1, 1)],
            idx.at[b], sems_i[b])
        return cv, ci

    pend = start(0, 0)
    for ch in range(NCH):
        b = ch % 2
        pend[0].wait()
        pend[1].wait()
        if ch + 1 < NCH:
            pend = start(ch + 1, (ch + 1) % 2)

        def et_body(et, carry):
            for l in range(8):
                iv = idx[b, et, 0, pl.ds(l * 16, 16)]
                for f in range(2):
                    vv = vals[b, et, f, pl.ds(l * 16, 16)]
                    plsc.addupdate_scatter(acc.at[f], [iv], vv)
            return carry

        lax.fori_loop(0, ET_CH, et_body, 0)

    pltpu.sync_copy(acc, out_hbm.at[q, pl.ds(2 * fg, 2), pl.ds(0, N)])


BN = 1024
NB = pl.cdiv(N, BN)


def _mlp_body(nf_ref, part_ref,
              w1a0, w1b0, b10, w20, b20,
              w1a1, w1b1, b11, w21, b21,
              out_ref):
    hnT = part_ref[0] + part_ref[1] + part_ref[2] + part_ref[3]
    nf = nf_ref[...]
    dn = (((0,), (0,)), ((), ()))
    x = jnp.dot(nf, w1a0[...], preferred_element_type=jnp.float32)
    x = x + lax.dot_general(hnT, w1b0[...], dn, preferred_element_type=jnp.float32)
    x = jnp.maximum(x + b10[...], 0.0)
    x = jnp.maximum(jnp.dot(x, w20[...], preferred_element_type=jnp.float32) + b20[...], 0.0)
    y = jnp.dot(x, w1a1[...], preferred_element_type=jnp.float32)
    y = y + lax.dot_general(hnT, w1b1[...], dn, preferred_element_type=jnp.float32)
    y = jnp.maximum(y + b11[...], 0.0)
    y = jnp.maximum(jnp.dot(y, w21[...], preferred_element_type=jnp.float32) + b21[...], 0.0)
    out_ref[...] = y


def _row_spec(d):
    return pl.BlockSpec((BN, d), lambda i: (i, 0))


def _full_spec(*shape):
    return pl.BlockSpec(shape, lambda i: (0,) * len(shape))


_mlp_call = pl.pallas_call(
    _mlp_body,
    grid=(NB,),
    in_specs=[
        _row_spec(D_IN),
        pl.BlockSpec((NQ, D_E, BN), lambda i: (0, 0, i)),
        _full_spec(D_IN, D_OUT), _full_spec(D_E, D_OUT), _full_spec(1, D_OUT),
        _full_spec(D_OUT, D_OUT), _full_spec(1, D_OUT),
        _full_spec(D_OUT, D_OUT), _full_spec(D_E, D_OUT), _full_spec(1, D_OUT),
        _full_spec(D_OUT, D_OUT), _full_spec(1, D_OUT),
    ],
    out_specs=_row_spec(D_OUT),
    out_shape=jax.ShapeDtypeStruct((N, D_OUT), jnp.float32),
)


@jax.jit
def kernel(nfeats, efeats, edge_index, W1_0, b1_0, W2_0, b2_0,
           W1_1, b1_1, W2_1, b2_1):
    ef2 = efeats.reshape(E, D_E).T.reshape(2, 8, ETILES, 128).transpose(0, 2, 1, 3)
    ei3 = edge_index.reshape(2, ETILES, 128).transpose(1, 0, 2)

    part = _sc_segment_sum(ef2, ei3)

    nf = nfeats.reshape(N, D_IN)
    out = _mlp_call(
        nf, part,
        W1_0[:D_IN], W1_0[D_IN:], b1_0.reshape(1, D_OUT),
        W2_0, b2_0.reshape(1, D_OUT),
        W1_1[:D_OUT], W1_1[D_OUT:], b1_1.reshape(1, D_OUT),
        W2_1, b2_1.reshape(1, D_OUT),
    )
    return out
